# baseline (speedup 1.0000x reference)
import functools

import jax
import jax.numpy as jnp
from jax import lax
from jax.experimental import pallas as pl
from jax.experimental.pallas import tpu as pltpu
from jax.experimental.pallas import tpu_sc as plsc

OUT_DIM = 64
VOCAB = 1_000_000
LANES = 16


def _embed_body(x_hbm, w_hbm, out_hbm, x_v, idx_v, out_v, sem):
    c = lax.axis_index("c")
    s = lax.axis_index("s")

    @pl.when(jnp.logical_and(c == 0, s == 0))
    def _():
        pltpu.sync_copy(x_hbm, x_v)
        xv = x_v[...]
        lanes = lax.iota(jnp.int32, 16)
        for g in range(OUT_DIM // LANES):
            idx_v[pl.ds(g * LANES, LANES)] = (
                lanes * VOCAB + (g * LANES * VOCAB) + xv
            )
        pltpu.async_copy(w_hbm.at[idx_v], out_v, sem).wait()
        pltpu.sync_copy(out_v, out_hbm)


_embed = functools.partial(
    pl.kernel,
    out_type=jax.ShapeDtypeStruct((OUT_DIM,), jnp.float32),
    mesh=plsc.VectorSubcoreMesh(core_axis_name="c", subcore_axis_name="s"),
    scratch_types=[
        pltpu.VMEM((LANES,), jnp.int32),
        pltpu.VMEM((OUT_DIM,), jnp.int32),
        pltpu.VMEM((OUT_DIM,), jnp.float32),
        pltpu.SemaphoreType.DMA,
    ],
)(_embed_body)


@jax.jit
def kernel(x, W):
    x16 = jnp.broadcast_to(x.astype(jnp.int32).reshape(()), (LANES,))
    w_flat = W.reshape(OUT_DIM * VOCAB)
    return _embed(x16, w_flat)


# speedup vs baseline: 242.0042x; 242.0042x over previous
import functools

import jax
import jax.numpy as jnp
from jax import lax
from jax.experimental import pallas as pl
from jax.experimental.pallas import tpu as pltpu
from jax.experimental.pallas import tpu_sc as plsc

OUT_DIM = 64
VOCAB = 1_000_000
LANES = 16
TILE_R = 8
TILE_C = 128


def _embed_body(x_hbm, w_hbm, out_hbm, x_v, buf_v, out_v, sem):
    c = lax.axis_index("c")
    s = lax.axis_index("s")

    @pl.when(jnp.logical_and(c == 0, s == 0))
    def _():
        pltpu.sync_copy(x_hbm, x_v)
        xs = x_v[...][0]
        base = pl.multiple_of(xs & -TILE_C, TILE_C)
        off = xs & (TILE_C - 1)
        copies = [
            pltpu.async_copy(
                w_hbm.at[pl.ds(t * TILE_R, TILE_R), pl.ds(base, TILE_C)],
                buf_v.at[pl.ds(t * TILE_R, TILE_R), :],
                sem,
            )
            for t in range(OUT_DIM // TILE_R)
        ]
        for cp in copies:
            cp.wait()
        lanes = lax.iota(jnp.int32, LANES)
        off16 = pl.multiple_of(off & -LANES, LANES)
        offmod = jnp.full((LANES,), off & (LANES - 1), dtype=jnp.int32)
        for g in range(OUT_DIM // LANES):
            acc = jnp.zeros((LANES,), jnp.float32)
            for j in range(LANES):
                v = buf_v[g * LANES + j, pl.ds(off16, LANES)]
                picked = lax.gather(
                    v,
                    offmod[:, None],
                    lax.GatherDimensionNumbers(
                        offset_dims=(),
                        collapsed_slice_dims=(0,),
                        start_index_map=(0,),
                    ),
                    slice_sizes=(1,),
                    mode=lax.GatherScatterMode.PROMISE_IN_BOUNDS,
                )
                acc = jnp.where(lanes == j, picked, acc)
            out_v[pl.ds(g * LANES, LANES)] = acc
        pltpu.sync_copy(out_v, out_hbm)


_embed = functools.partial(
    pl.kernel,
    out_type=jax.ShapeDtypeStruct((OUT_DIM,), jnp.float32),
    mesh=plsc.VectorSubcoreMesh(core_axis_name="c", subcore_axis_name="s"),
    scratch_types=[
        pltpu.VMEM((LANES,), jnp.int32),
        pltpu.VMEM((OUT_DIM, TILE_C), jnp.float32),
        pltpu.VMEM((OUT_DIM,), jnp.float32),
        pltpu.SemaphoreType.DMA,
    ],
)(_embed_body)


@jax.jit
def kernel(x, W):
    x16 = jnp.broadcast_to(x.astype(jnp.int32).reshape(()), (LANES,))
    return _embed(x16, W)


# speedup vs baseline: 248.4042x; 1.0264x over previous
import functools

import jax
import jax.numpy as jnp
from jax import lax
from jax.experimental import pallas as pl
from jax.experimental.pallas import tpu as pltpu
from jax.experimental.pallas import tpu_sc as plsc

OUT_DIM = 64
VOCAB = 1_000_000
LANES = 16
TILE_R = 8
TILE_C = 128


def _embed_body(x_hbm, w_hbm, out_hbm, x_v, buf_v, out_v, sem):
    c = lax.axis_index("c")
    s = lax.axis_index("s")

    @pl.when(jnp.logical_and(c == 0, s == 0))
    def _():
        pltpu.sync_copy(x_hbm, x_v.at[pl.ds(0, 1)])
        xs = x_v[...][0]
        base = pl.multiple_of(xs & -TILE_C, TILE_C)
        off = xs & (TILE_C - 1)
        copies = [
            pltpu.async_copy(
                w_hbm.at[pl.ds(t * TILE_R, TILE_R), pl.ds(base, TILE_C)],
                buf_v.at[pl.ds(t * TILE_R, TILE_R), :],
                sem,
            )
            for t in range(OUT_DIM // TILE_R)
        ]
        for cp in copies:
            cp.wait()
        lanes = lax.iota(jnp.int32, LANES)
        off16 = pl.multiple_of(off & -LANES, LANES)
        offmod = jnp.full((LANES,), off & (LANES - 1), dtype=jnp.int32)

        def gbody(g, carry):
            acc = jnp.zeros((LANES,), jnp.float32)
            for j in range(LANES):
                v = buf_v[g * LANES + j, pl.ds(off16, LANES)]
                picked = lax.gather(
                    v,
                    offmod[:, None],
                    lax.GatherDimensionNumbers(
                        offset_dims=(),
                        collapsed_slice_dims=(0,),
                        start_index_map=(0,),
                    ),
                    slice_sizes=(1,),
                    mode=lax.GatherScatterMode.PROMISE_IN_BOUNDS,
                )
                acc = jnp.where(lanes == j, picked, acc)
            out_v[pl.ds(pl.multiple_of(g * LANES, LANES), LANES)] = acc
            return carry

        lax.fori_loop(0, OUT_DIM // LANES, gbody, 0)
        pltpu.sync_copy(out_v, out_hbm)


_embed = functools.partial(
    pl.kernel,
    out_type=jax.ShapeDtypeStruct((OUT_DIM,), jnp.float32),
    mesh=plsc.VectorSubcoreMesh(core_axis_name="c", subcore_axis_name="s"),
    scratch_types=[
        pltpu.VMEM((LANES,), jnp.int32),
        pltpu.VMEM((OUT_DIM, TILE_C), jnp.float32),
        pltpu.VMEM((OUT_DIM,), jnp.float32),
        pltpu.SemaphoreType.DMA,
    ],
)(_embed_body)


@jax.jit
def kernel(x, W):
    x1 = x.astype(jnp.int32).reshape((1,))
    return _embed(x1, W)


# speedup vs baseline: 264.7800x; 1.0659x over previous
import functools

import jax
import jax.numpy as jnp
from jax import lax
from jax.experimental import pallas as pl
from jax.experimental.pallas import tpu as pltpu
from jax.experimental.pallas import tpu_sc as plsc

OUT_DIM = 64
VOCAB = 1_000_000
LANES = 16
TILE_R = 8
TILE_C = 128


def _embed_body(x_hbm, w_hbm, out_hbm, x_v, buf_v, out_v, sem):
    c = lax.axis_index("c")
    s = lax.axis_index("s")

    @pl.when(jnp.logical_and(c == 0, s == 0))
    def _():
        pltpu.sync_copy(x_hbm, x_v.at[pl.ds(0, 1)])
        xs = x_v[...][0]
        base = pl.multiple_of(xs & -TILE_C, TILE_C)
        off = xs & (TILE_C - 1)
        pltpu.async_copy(
            w_hbm.at[:, pl.ds(base, TILE_C)], buf_v, sem
        ).wait()
        lanes = lax.iota(jnp.int32, LANES)
        off16 = pl.multiple_of(off & -LANES, LANES)
        offmod = jnp.full((LANES,), off & (LANES - 1), dtype=jnp.int32)

        def gbody(g, carry):
            acc = jnp.zeros((LANES,), jnp.float32)
            for j in range(LANES):
                v = buf_v[g * LANES + j, pl.ds(off16, LANES)]
                picked = lax.gather(
                    v,
                    offmod[:, None],
                    lax.GatherDimensionNumbers(
                        offset_dims=(),
                        collapsed_slice_dims=(0,),
                        start_index_map=(0,),
                    ),
                    slice_sizes=(1,),
                    mode=lax.GatherScatterMode.PROMISE_IN_BOUNDS,
                )
                acc = jnp.where(lanes == j, picked, acc)
            out_v[pl.ds(pl.multiple_of(g * LANES, LANES), LANES)] = acc
            return carry

        lax.fori_loop(0, OUT_DIM // LANES, gbody, 0)
        pltpu.sync_copy(out_v, out_hbm)


_embed = functools.partial(
    pl.kernel,
    out_type=jax.ShapeDtypeStruct((OUT_DIM,), jnp.float32),
    mesh=plsc.VectorSubcoreMesh(
        core_axis_name="c", subcore_axis_name="s", num_cores=1
    ),
    scratch_types=[
        pltpu.VMEM((LANES,), jnp.int32),
        pltpu.VMEM((OUT_DIM, TILE_C), jnp.float32),
        pltpu.VMEM((OUT_DIM,), jnp.float32),
        pltpu.SemaphoreType.DMA,
    ],
)(_embed_body)


@jax.jit
def kernel(x, W):
    x1 = x.astype(jnp.int32).reshape((1,))
    return _embed(x1, W)


# speedup vs baseline: 271.6829x; 1.0261x over previous
import functools

import jax
import jax.numpy as jnp
from jax import lax
from jax.experimental import pallas as pl
from jax.experimental.pallas import tpu as pltpu
from jax.experimental.pallas import tpu_sc as plsc

OUT_DIM = 64
VOCAB = 1_000_000
LANES = 16
TILE_C = 128
NTILES = OUT_DIM // LANES


def _embed_body(x_hbm, w_hbm, out_hbm, x_v, buf_v, out_v, sem):
    c = lax.axis_index("c")
    s = lax.axis_index("s")

    @pl.when(jnp.logical_and(c == 0, s < NTILES))
    def _():
        pltpu.sync_copy(x_hbm, x_v.at[pl.ds(0, 1)])
        xs = x_v[...][0]
        base = pl.multiple_of(xs & -TILE_C, TILE_C)
        off = xs & (TILE_C - 1)
        row0 = pl.multiple_of(s * LANES, LANES)
        pltpu.async_copy(
            w_hbm.at[pl.ds(row0, LANES), pl.ds(base, TILE_C)], buf_v, sem
        ).wait()
        lanes = lax.iota(jnp.int32, LANES)
        off16 = pl.multiple_of(off & -LANES, LANES)
        offmod = jnp.full((LANES,), off & (LANES - 1), dtype=jnp.int32)
        acc = jnp.zeros((LANES,), jnp.float32)
        for j in range(LANES):
            v = buf_v[j, pl.ds(off16, LANES)]
            picked = lax.gather(
                v,
                offmod[:, None],
                lax.GatherDimensionNumbers(
                    offset_dims=(),
                    collapsed_slice_dims=(0,),
                    start_index_map=(0,),
                ),
                slice_sizes=(1,),
                mode=lax.GatherScatterMode.PROMISE_IN_BOUNDS,
            )
            acc = jnp.where(lanes == j, picked, acc)
        out_v[...] = acc
        pltpu.sync_copy(out_v, out_hbm.at[pl.ds(row0, LANES)])


_embed = functools.partial(
    pl.kernel,
    out_type=jax.ShapeDtypeStruct((OUT_DIM,), jnp.float32),
    mesh=plsc.VectorSubcoreMesh(
        core_axis_name="c", subcore_axis_name="s", num_cores=1
    ),
    scratch_types=[
        pltpu.VMEM((LANES,), jnp.int32),
        pltpu.VMEM((LANES, TILE_C), jnp.float32),
        pltpu.VMEM((LANES,), jnp.float32),
        pltpu.SemaphoreType.DMA,
    ],
)(_embed_body)


@jax.jit
def kernel(x, W):
    x1 = x.astype(jnp.int32).reshape((1,))
    return _embed(x1, W)
